# Initial kernel scaffold; baseline (speedup 1.0000x reference)
import jax
import jax.numpy as jnp
from jax.experimental import pallas as pl


def kernel(ind, embed_weight):
    raise NotImplementedError("write your pallas kernel here")


# speedup vs baseline: 3.2774x; 3.2774x over previous
import functools

import jax
import jax.numpy as jnp
from jax import lax
from jax.experimental import pallas as pl
from jax.experimental.pallas import tpu as pltpu
from jax.experimental.pallas import tpu_sc as plsc

DIM = 256
NC, NS = 2, 16
NW = NC * NS
B = 32 * 1024
B_PER_W = B // NW
CHUNK = 128
N_CHUNKS = B_PER_W // CHUNK


@functools.cache
def _build():
    mesh = plsc.VectorSubcoreMesh(core_axis_name="c", subcore_axis_name="s")

    @functools.partial(
        pl.kernel,
        mesh=mesh,
        out_type=jax.ShapeDtypeStruct((B, DIM), jnp.float32),
        scratch_types=[
            pltpu.VMEM((N_CHUNKS, CHUNK), jnp.int32),
            pltpu.VMEM((2, CHUNK, DIM), jnp.float32),
            pltpu.SemaphoreType.DMA,
            pltpu.SemaphoreType.DMA,
        ],
    )
    def gather_kernel(idx_hbm, table_hbm, out_hbm, idx_v, rows_v, gsem, ssem):
        wid = lax.axis_index("s") * NC + lax.axis_index("c")
        base = wid * B_PER_W
        pltpu.sync_copy(idx_hbm.at[wid], idx_v)
        gathers = {}
        scatters = {}
        gathers[0] = pltpu.async_copy(table_hbm.at[idx_v.at[0]], rows_v.at[0], gsem)
        for j in range(N_CHUNKS):
            slot = j % 2
            if j + 1 < N_CHUNKS:
                if j >= 1:
                    scatters[j - 1].wait()
                gathers[j + 1] = pltpu.async_copy(
                    table_hbm.at[idx_v.at[j + 1]], rows_v.at[1 - slot], gsem)
            gathers[j].wait()
            scatters[j] = pltpu.async_copy(
                rows_v.at[slot], out_hbm.at[pl.ds(base + j * CHUNK, CHUNK)], ssem)
        scatters[N_CHUNKS - 2].wait()
        scatters[N_CHUNKS - 1].wait()

    return gather_kernel


def kernel(ind, embed_weight):
    idx = ind.reshape(NW, N_CHUNKS, CHUNK)
    out = _build()(idx, embed_weight)
    return out.reshape(ind.shape[0], ind.shape[1], DIM)
